# Initial kernel scaffold; baseline (speedup 1.0000x reference)
import jax
import jax.numpy as jnp
from jax.experimental import pallas as pl


def kernel(situation_x, cmd_h, cmd_out, edge_index, cmdLength, graph_membership, Wmap, W1, W2, W3, b3, W4, W5, W6, W7, W8, W9, W10, W11, W11b, W12, initMem):
    raise NotImplementedError("write your pallas kernel here")


# speedup vs baseline: 1.0092x; 1.0092x over previous
import functools

import jax
import jax.numpy as jnp
from jax import lax
from jax.experimental import pallas as pl
from jax.experimental.pallas import tpu as pltpu

N = 10000
E = 160000
B = 64
L = 20
D = 256
T = 4

_RB = 1000


def _dotb(a, b):
    return jnp.dot(a.astype(jnp.bfloat16), b.astype(jnp.bfloat16),
                   preferred_element_type=jnp.float32)


def _k0_body(cmd_h_ref, cmd_out_ref, mask_ref, w3_ref, b3_ref, w2_ref, w1_ref,
             out_ref):
    f32 = jnp.float32
    bf16 = jnp.bfloat16
    r = jax.nn.relu(_dotb(cmd_h_ref[...], w3_ref[...]) + b3_ref[...])
    cmd_out = cmd_out_ref[...]
    cob = cmd_out.astype(bf16).astype(f32)
    mask = mask_ref[...]
    w1b = w1_ref[...].astype(bf16).astype(f32)
    for t in range(T):
        qt = _dotb(r, w2_ref[t])
        p = (cmd_out * qt[:, None, :]).astype(bf16).astype(f32)
        raw = jnp.sum(p * w1b[None, :, :], axis=-1)
        raw = jnp.where(mask > 0, raw, -1e30)
        att = jax.nn.softmax(raw, axis=-1)
        attb = att.astype(bf16).astype(f32)
        out_ref[t] = jnp.sum(attb[:, :, None] * cob, axis=1)


def _compute_cmd_all(cmd_h, cmd_out, cmdLength, W1, W2, W3, b3):
    mask = (jnp.arange(L)[None, :] < cmdLength[:, None]).astype(jnp.float32)
    return pl.pallas_call(
        _k0_body,
        out_shape=jax.ShapeDtypeStruct((T, B, D), jnp.float32),
    )(cmd_h, cmd_out, mask, W3, b3.reshape(1, D), W2, W1.reshape(1, D))


def _k1_body(xl_ref, xc_ref, s_ref, oh_ref, cmd_ref,
             w11_ref, w11b_ref, w4_ref, w5_ref,
             w6a_ref, w6b_ref, w6c_ref,
             w7a_ref, w7b_ref, w7c_ref, w8_ref,
             w9a_ref, w9b_ref, w9c_ref, w10_ref,
             xcn_ref, srcctx_ref, dstctx_ref, ft_ref):
    dot = _dotb
    xl = xl_ref[...]
    xc = dot(xc_ref[...], w11_ref[...]) + dot(s_ref[...], w11b_ref[...])
    xcn_ref[...] = xc
    cb = dot(oh_ref[...], cmd_ref[...])
    fuse = dot(xl, w4_ref[...]) * dot(xc, w5_ref[...])
    cat6 = dot(xl, w6a_ref[...]) + dot(xc, w6b_ref[...]) + dot(fuse, w6c_ref[...])
    cat7 = dot(xl, w7a_ref[...]) + dot(xc, w7b_ref[...]) + dot(fuse, w7c_ref[...])
    cat9 = dot(xl, w9a_ref[...]) + dot(xc, w9b_ref[...]) + dot(fuse, w9c_ref[...])
    srcctx_ref[...] = cat7 * dot(cb, w8_ref[...])
    dstctx_ref[...] = cat6
    ft_ref[...] = cat9 * dot(cb, w10_ref[...])


def _dense_stage(x_loc, x_ctx, s_prev, onehot, cmd_t, W11, W11b, W4, W5, W6,
                 W7, W8, W9, W10):
    nb = N // _RB
    row_spec = pl.BlockSpec((_RB, D), lambda i: (i, 0))
    oh_spec = pl.BlockSpec((_RB, B), lambda i: (i, 0))
    full = lambda shape: pl.BlockSpec(shape, lambda i: tuple(0 for _ in shape))
    w_spec = full((D, D))
    w6a, w6b, w6c = W6[:D], W6[D:2 * D], W6[2 * D:]
    w7a, w7b, w7c = W7[:D], W7[D:2 * D], W7[2 * D:]
    w9a, w9b, w9c = W9[:D], W9[D:2 * D], W9[2 * D:]
    out_shapes = [jax.ShapeDtypeStruct((N, D), jnp.float32) for _ in range(4)]
    return pl.pallas_call(
        _k1_body,
        grid=(nb,),
        in_specs=[row_spec, row_spec, row_spec, oh_spec, full((B, D))]
        + [w_spec] * 15,
        out_specs=[row_spec] * 4,
        out_shape=out_shapes,
    )(x_loc, x_ctx, s_prev, onehot, cmd_t, W11, W11b, W4, W5,
      w6a, w6b, w6c, w7a, w7b, w7c, W8, w9a, w9b, w9c, W10)


def _k3_body(xl_ref, xc_ref, s_ref, w11_ref, w11b_ref, w12a_ref, w12b_ref,
             out_ref):
    dot = _dotb
    xc = dot(xc_ref[...], w11_ref[...]) + dot(s_ref[...], w11b_ref[...])
    out_ref[...] = dot(xl_ref[...], w12a_ref[...]) + dot(xc, w12b_ref[...])


def _final_stage(x_loc, x_ctx, s_prev, W11, W11b, W12):
    nb = N // _RB
    row_spec = pl.BlockSpec((_RB, D), lambda i: (i, 0))
    w_spec = pl.BlockSpec((D, D), lambda i: (0, 0))
    return pl.pallas_call(
        _k3_body,
        grid=(nb,),
        in_specs=[row_spec] * 3 + [w_spec] * 4,
        out_specs=row_spec,
        out_shape=jax.ShapeDtypeStruct((N, D), jnp.float32),
    )(x_loc, x_ctx, s_prev, W11, W11b, W12[:D], W12[D:])


def _matmul_body(x_ref, w_ref, o_ref):
    o_ref[...] = _dotb(x_ref[...], w_ref[...])


def _matmul(x, w):
    nb = N // _RB
    return pl.pallas_call(
        _matmul_body,
        grid=(nb,),
        in_specs=[pl.BlockSpec((_RB, D), lambda i: (i, 0)),
                  pl.BlockSpec((D, D), lambda i: (0, 0))],
        out_specs=pl.BlockSpec((_RB, D), lambda i: (i, 0)),
        out_shape=jax.ShapeDtypeStruct((N, D), jnp.float32),
    )(x, w)


def _edge_stage(src, dst, src_ctx, dst_ctx, ft):
    e = jnp.sum(jnp.take(src_ctx, src, axis=0) * jnp.take(dst_ctx, dst, axis=0),
                axis=-1)
    mseg = jax.ops.segment_max(e, dst, num_segments=N)
    mseg = jnp.where(jnp.isfinite(mseg), mseg, 0.0)
    ee = jnp.exp(e - jnp.take(mseg, dst))
    denom = jax.ops.segment_sum(ee, dst, num_segments=N)
    a = ee / (jnp.take(denom, dst) + 1e-9)
    return jax.ops.segment_sum(jnp.take(ft, src, axis=0) * a[:, None], dst,
                               num_segments=N)


def kernel(situation_x, cmd_h, cmd_out, edge_index, cmdLength,
           graph_membership, Wmap, W1, W2, W3, b3, W4, W5, W6, W7, W8, W9,
           W10, W11, W11b, W12, initMem):
    f32 = jnp.float32
    src = edge_index[0]
    dst = edge_index[1]
    onehot = (graph_membership[:, None] == jnp.arange(B)[None, :]).astype(f32)

    cmd_all = _compute_cmd_all(cmd_h, cmd_out, cmdLength, W1, W2, W3, b3)
    x_loc = _matmul(situation_x, Wmap)

    x_ctx = jnp.broadcast_to(initMem, (N, D))
    s_prev = jnp.zeros((N, D), f32)
    w11_t = jnp.eye(D, dtype=f32)
    w11b_t = jnp.zeros((D, D), f32)
    for t in range(T):
        x_ctx, src_ctx, dst_ctx, ft = _dense_stage(
            x_loc, x_ctx, s_prev, onehot, cmd_all[t], w11_t, w11b_t,
            W4, W5, W6, W7, W8, W9, W10)
        s_prev = _edge_stage(src, dst, src_ctx, dst_ctx, ft)
        w11_t, w11b_t = W11, W11b
    return _final_stage(x_loc, x_ctx, s_prev, W11, W11b, W12)


# speedup vs baseline: 1.7372x; 1.7213x over previous
import dataclasses
import functools

import jax
import jax.numpy as jnp
from jax import lax
from jax.experimental import pallas as pl
from jax.experimental.pallas import tpu as pltpu

N = 10000
E = 160000
B = 64
L = 20
D = 256
T = 4

_RB = 1000


def _dotb(a, b):
    return jnp.dot(a.astype(jnp.bfloat16), b.astype(jnp.bfloat16),
                   preferred_element_type=jnp.float32)


def _k0_body(cmd_h_ref, cmd_out_ref, mask_ref, w3_ref, b3_ref, w2_ref, w1_ref,
             out_ref):
    f32 = jnp.float32
    bf16 = jnp.bfloat16
    r = jax.nn.relu(_dotb(cmd_h_ref[...], w3_ref[...]) + b3_ref[...])
    cmd_out = cmd_out_ref[...]
    cob = cmd_out.astype(bf16).astype(f32)
    mask = mask_ref[...]
    w1b = w1_ref[...].astype(bf16).astype(f32)
    for t in range(T):
        qt = _dotb(r, w2_ref[t])
        p = (cmd_out * qt[:, None, :]).astype(bf16).astype(f32)
        raw = jnp.sum(p * w1b[None, :, :], axis=-1)
        raw = jnp.where(mask > 0, raw, -1e30)
        att = jax.nn.softmax(raw, axis=-1)
        attb = att.astype(bf16).astype(f32)
        out_ref[t] = jnp.sum(attb[:, :, None] * cob, axis=1)


def _compute_cmd_all(cmd_h, cmd_out, cmdLength, W1, W2, W3, b3):
    mask = (jnp.arange(L)[None, :] < cmdLength[:, None]).astype(jnp.float32)
    return pl.pallas_call(
        _k0_body,
        out_shape=jax.ShapeDtypeStruct((T, B, D), jnp.float32),
    )(cmd_h, cmd_out, mask, W3, b3.reshape(1, D), W2, W1.reshape(1, D))


def _k1_body(xl_ref, xc_ref, s_ref, oh_ref, cmd_ref,
             w11_ref, w11b_ref, w4_ref, w5_ref,
             w6a_ref, w6b_ref, w6c_ref,
             w7a_ref, w7b_ref, w7c_ref, w8_ref,
             w9a_ref, w9b_ref, w9c_ref, w10_ref,
             xcn_ref, srcctx_ref, dstctx_ref, ft_ref):
    dot = _dotb
    xl = xl_ref[...]
    xc = dot(xc_ref[...], w11_ref[...]) + dot(s_ref[...], w11b_ref[...])
    xcn_ref[...] = xc
    cb = dot(oh_ref[...], cmd_ref[...])
    fuse = dot(xl, w4_ref[...]) * dot(xc, w5_ref[...])
    cat6 = dot(xl, w6a_ref[...]) + dot(xc, w6b_ref[...]) + dot(fuse, w6c_ref[...])
    cat7 = dot(xl, w7a_ref[...]) + dot(xc, w7b_ref[...]) + dot(fuse, w7c_ref[...])
    cat9 = dot(xl, w9a_ref[...]) + dot(xc, w9b_ref[...]) + dot(fuse, w9c_ref[...])
    srcctx_ref[...] = cat7 * dot(cb, w8_ref[...])
    dstctx_ref[...] = cat6
    ft_ref[...] = cat9 * dot(cb, w10_ref[...])


def _dense_stage(x_loc, x_ctx, s_prev, onehot, cmd_t, W11, W11b, W4, W5, W6,
                 W7, W8, W9, W10):
    nb = N // _RB
    row_spec = pl.BlockSpec((_RB, D), lambda i: (i, 0))
    oh_spec = pl.BlockSpec((_RB, B), lambda i: (i, 0))
    full = lambda shape: pl.BlockSpec(shape, lambda i: tuple(0 for _ in shape))
    w_spec = full((D, D))
    w6a, w6b, w6c = W6[:D], W6[D:2 * D], W6[2 * D:]
    w7a, w7b, w7c = W7[:D], W7[D:2 * D], W7[2 * D:]
    w9a, w9b, w9c = W9[:D], W9[D:2 * D], W9[2 * D:]
    out_shapes = [jax.ShapeDtypeStruct((N, D), jnp.float32) for _ in range(4)]
    return pl.pallas_call(
        _k1_body,
        grid=(nb,),
        in_specs=[row_spec, row_spec, row_spec, oh_spec, full((B, D))]
        + [w_spec] * 15,
        out_specs=[row_spec] * 4,
        out_shape=out_shapes,
    )(x_loc, x_ctx, s_prev, onehot, cmd_t, W11, W11b, W4, W5,
      w6a, w6b, w6c, w7a, w7b, w7c, W8, w9a, w9b, w9c, W10)


def _k3_body(xl_ref, xc_ref, s_ref, w11_ref, w11b_ref, w12a_ref, w12b_ref,
             out_ref):
    dot = _dotb
    xc = dot(xc_ref[...], w11_ref[...]) + dot(s_ref[...], w11b_ref[...])
    out_ref[...] = dot(xl_ref[...], w12a_ref[...]) + dot(xc, w12b_ref[...])


def _final_stage(x_loc, x_ctx, s_prev, W11, W11b, W12):
    nb = N // _RB
    row_spec = pl.BlockSpec((_RB, D), lambda i: (i, 0))
    w_spec = pl.BlockSpec((D, D), lambda i: (0, 0))
    return pl.pallas_call(
        _k3_body,
        grid=(nb,),
        in_specs=[row_spec] * 3 + [w_spec] * 4,
        out_specs=row_spec,
        out_shape=jax.ShapeDtypeStruct((N, D), jnp.float32),
    )(x_loc, x_ctx, s_prev, W11, W11b, W12[:D], W12[D:])


def _matmul_body(x_ref, w_ref, o_ref):
    o_ref[...] = _dotb(x_ref[...], w_ref[...])


def _matmul(x, w):
    nb = N // _RB
    return pl.pallas_call(
        _matmul_body,
        grid=(nb,),
        in_specs=[pl.BlockSpec((_RB, D), lambda i: (i, 0)),
                  pl.BlockSpec((D, D), lambda i: (0, 0))],
        out_specs=pl.BlockSpec((_RB, D), lambda i: (i, 0)),
        out_shape=jax.ShapeDtypeStruct((N, D), jnp.float32),
    )(x, w)


from jax.experimental.pallas import tpu_sc as plsc

_NC, _NS, _LN = 2, 16, 16
_NW = _NC * _NS
_NBLK = N // 16
_CT = E // 16 + N
_ECAP = 8192
_CTSL = 1040
_SRCSL = _ECAP + 16
_NSCAP = N + 8
_KB = 8


def _edge_preprocess(edge_index):
    i32 = jnp.int32
    src = edge_index[0]
    dst = edge_index[1]
    order = jnp.argsort(dst)
    src_s = src[order].astype(i32)
    dst_s = dst[order].astype(i32)
    ns = jnp.searchsorted(dst_s, jnp.arange(N + 1, dtype=i32),
                          side='left').astype(i32)
    deg = ns[1:] - ns[:-1]
    cns = jnp.concatenate([jnp.zeros((1,), i32),
                           jnp.cumsum((deg + 15) // 16).astype(i32)])
    cidx = jnp.arange(_CT, dtype=i32)
    nofc = jnp.clip(jnp.searchsorted(cns, cidx, side='right') - 1,
                    0, N - 1).astype(i32)
    k16 = (cidx - cns[nofc]) * 16
    cst = jnp.clip(ns[nofc] + k16, 0, E - 1).astype(i32)
    cln = jnp.clip(deg[nofc] - k16, 0, 16).astype(i32)
    tgt = (jnp.arange(_NW + 1, dtype=i32) * (E // _NW)).astype(i32)
    nb_ = jnp.searchsorted(ns, tgt, side='left').astype(i32)
    blk = jnp.clip((nb_ + 8) // 16, 0, _NBLK)
    blk = blk.at[0].set(0).at[_NW].set(_NBLK)
    blk = jax.lax.cummax(blk)
    tcs = cns[16 * blk]
    teb = ns[16 * blk]
    src_pad = jnp.pad(src_s, (0, _SRCSL + 16))
    ns_pad = jnp.pad(ns, (0, _NSCAP - (N + 1)), constant_values=E)
    cns_pad = jnp.pad(cns, (0, _NSCAP - (N + 1)), mode='edge')
    pad_ct = lambda a, v: jnp.pad(a, (0, _CTSL + 16), constant_values=v)
    pad_t = lambda a: jnp.pad(a.astype(i32), (0, 7), mode='edge')
    return (src_pad, ns_pad, cns_pad, pad_ct(cst, 0), pad_ct(nofc, N - 1),
            pad_ct(cln, 0), pad_t(blk), pad_t(tcs), pad_t(teb))


def _lane(v, j, il):
    return jnp.sum(jnp.where(il == j, v, jnp.zeros_like(v)))


def _edge_sc_body(srcctx, dstctx, ftab, srcpad, nsh, cnsh, csth, cndh, clnh,
                  tblkh, tcsh, tebh, out_s,
                  nsbuf, cnsbuf, srcbuf, cstbuf, cndbuf, clnbuf, ebuf, sbuf,
                  dbuf, idxb, stg0, stg1, tblk_v, tcs_v, teb_v,
                  semA, semF0, semF1):
    i32 = jnp.int32
    f32 = jnp.float32
    il = lax.iota(i32, 16)
    wid = lax.axis_index("s") * _NC + lax.axis_index("c")

    d0 = pltpu.async_copy(tblkh, tblk_v, semA)
    d1 = pltpu.async_copy(tcsh, tcs_v, semA)
    d2 = pltpu.async_copy(tebh, teb_v, semA)
    d0.wait(); d1.wait(); d2.wait()
    widx = jnp.clip(jnp.full((16,), wid, jnp.int32) + il, 0, _NW)
    blkg = plsc.load_gather(tblk_v, [widx])
    tcsg = plsc.load_gather(tcs_v, [widx])
    tebg = plsc.load_gather(teb_v, [widx])
    blk0 = _lane(blkg, 0, il)
    blk1 = _lane(blkg, 1, il)
    tcs0 = _lane(tcsg, 0, il)
    tcs1 = _lane(tcsg, 1, il)
    teb0 = _lane(tebg, 0, il)
    ebase = (teb0 // 8) * 8
    ctbase = (tcs0 // 8) * 8
    ds = [pltpu.async_copy(nsh, nsbuf, semA),
          pltpu.async_copy(cnsh, cnsbuf, semA),
          pltpu.async_copy(srcpad.at[pl.ds(ebase, _SRCSL)], srcbuf, semA),
          pltpu.async_copy(csth.at[pl.ds(ctbase, _CTSL)], cstbuf, semA),
          pltpu.async_copy(cndh.at[pl.ds(ctbase, _CTSL)], cndbuf, semA),
          pltpu.async_copy(clnh.at[pl.ds(ctbase, _CTSL)], clnbuf, semA)]
    for d in ds:
        d.wait()

    def abatch(bo, _):
        p0 = tcs0 + _KB * bo
        idxl = jnp.clip(jnp.full((16,), p0 - ctbase, i32) + il, 0, _CTSL - 1)
        cs16 = plsc.load_gather(cstbuf, [idxl])
        cn16 = plsc.load_gather(cndbuf, [idxl])
        cl16 = plsc.load_gather(clnbuf, [idxl])
        descs = []
        csl = []
        cll = []
        for b in range(_KB):
            cs_b = _lane(cs16, b, il)
            cn_b = jnp.clip(_lane(cn16, b, il), 0, N - 1)
            cl_b = _lane(cl16, b, il)
            si = jnp.clip(jnp.full((16,), cs_b - ebase, i32) + il,
                          0, _SRCSL - 1)
            sidx = jnp.clip(plsc.load_gather(srcbuf, [si]), 0, N - 1)
            idxb[b, pl.ds(0, 16)] = sidx
            descs.append(pltpu.async_copy(srcctx.at[idxb.at[b]], sbuf.at[b],
                                          semA))
            descs.append(pltpu.async_copy(dstctx.at[cn_b], dbuf.at[b], semA))
            csl.append(cs_b)
            cll.append(cl_b)
        for d in descs:
            d.wait()
        for b in range(_KB):
            dr = [dbuf[b, pl.ds(16 * k, 16)] for k in range(16)]

            def jdot(j, ev, _b=b, _dr=dr):
                acc = sbuf[_b, j, pl.ds(0, 16)] * _dr[0]
                for k in range(1, 16):
                    acc = acc + sbuf[_b, j, pl.ds(16 * k, 16)] * _dr[k]
                return jnp.where(il == j, jnp.sum(acc), ev)

            ev = lax.fori_loop(0, 16, jdot, jnp.zeros((16,), f32))
            st = jnp.clip(jnp.full((16,), csl[b] - teb0, i32) + il,
                          0, _ECAP - 1)
            plsc.store_scatter(ebuf, [st], ev, mask=il < cll[b])
        return 0

    lax.fori_loop(0, (tcs1 - tcs0 + _KB - 1) // _KB, abatch, 0)

    def nbody(v, _):
        nsg = plsc.load_gather(
            nsbuf, [jnp.clip(jnp.full((16,), v, i32) + il, 0, _NSCAP - 1)])
        s0 = _lane(nsg, 0, il)
        s1 = _lane(nsg, 1, il)
        d = s1 - s0

        @pl.when(d > 0)
        def _():
            base = s0 - teb0
            nch = (d + 15) // 16

            def midx(c):
                ii = jnp.clip(jnp.full((16,), base + 16 * c, i32) + il,
                              0, _ECAP - 1)
                return ii, (16 * c + il) < d

            def mbody(c, m):
                ii, msk = midx(c)
                eg = plsc.load_gather(ebuf, [ii])
                return jnp.maximum(m, jnp.max(jnp.where(msk, eg, f32(-1e30))))

            m = lax.fori_loop(0, nch, mbody, f32(-1e30))

            def sbody(c, dn):
                ii, msk = midx(c)
                eg = plsc.load_gather(ebuf, [ii])
                ee = jnp.where(msk, jnp.exp(eg - m), f32(0))
                plsc.store_scatter(ebuf, [ii], ee, mask=msk)
                return dn + jnp.sum(ee)

            dn = lax.fori_loop(0, nch, sbody, f32(0))
            dv = jnp.full((16,), dn + f32(1e-9), f32)
            yi = (jnp.full((16,), jnp.int32(0x7EB53567), i32)
                  - plsc.bitcast(dv, i32))
            inv = plsc.bitcast(yi, f32)
            two = jnp.full((16,), f32(2), f32)
            for _ in range(5):
                inv = inv * (two - dv * inv)

            def pbody(c, _2):
                ii, msk = midx(c)
                eg = plsc.load_gather(ebuf, [ii])
                plsc.store_scatter(ebuf, [ii], eg * inv, mask=msk)
                return 0

            lax.fori_loop(0, nch, pbody, 0)
        return 0

    lax.fori_loop(16 * blk0, 16 * blk1, nbody, 0)

    def cblk(g, _):
        cb = plsc.load_gather(
            cnsbuf,
            [jnp.clip(jnp.full((16,), 16 * g, i32) + 16 * il, 0, _NSCAP - 1)])
        c0 = _lane(cb, 0, il)
        c1 = _lane(cb, 1, il)

        def do(stg, semF):
            @pl.when(g - 2 >= blk0)
            def _():
                pltpu.make_async_copy(
                    stg, out_s.at[pl.ds(16 * (g - 2), 16)], semF).wait()
            for r in range(16):
                for k in range(16):
                    stg[r, pl.ds(16 * k, 16)] = jnp.zeros((16,), f32)

            def cbatch(bo, _2):
                p0 = c0 + _KB * bo
                idxl = jnp.clip(jnp.full((16,), p0 - ctbase, i32) + il,
                                0, _CTSL - 1)
                cs16 = plsc.load_gather(cstbuf, [idxl])
                cn16 = plsc.load_gather(cndbuf, [idxl])
                cl16 = jnp.where((jnp.full((16,), p0, i32) + il) < c1,
                                 plsc.load_gather(clnbuf, [idxl]),
                                 jnp.zeros((16,), i32))
                descs = []
                afl = []
                rowl = []
                for b in range(_KB):
                    cs_b = _lane(cs16, b, il)
                    cn_b = _lane(cn16, b, il)
                    cl_b = _lane(cl16, b, il)
                    ei = jnp.clip(jnp.full((16,), cs_b - teb0, i32) + il,
                                  0, _ECAP - 1)
                    av = plsc.load_gather(ebuf, [ei])
                    af = jnp.where(il < cl_b, av, f32(0))
                    si = jnp.clip(jnp.full((16,), cs_b - ebase, i32) + il,
                                  0, _SRCSL - 1)
                    sidx = jnp.clip(plsc.load_gather(srcbuf, [si]), 0, N - 1)
                    idxb[b, pl.ds(0, 16)] = sidx
                    descs.append(pltpu.async_copy(ftab.at[idxb.at[b]],
                                                  sbuf.at[b], semA))
                    afl.append(af)
                    rowl.append(jnp.clip(cn_b - 16 * g, 0, 15))
                for d in descs:
                    d.wait()
                for b in range(_KB):
                    af = afl[b]

                    def jacc(j, accs, _b=b, _af=af):
                        aj = jnp.sum(jnp.where(il == j, _af,
                                               jnp.zeros((16,), f32)))
                        return tuple(
                            accs[k] + aj * sbuf[_b, j, pl.ds(16 * k, 16)]
                            for k in range(16))

                    accs = lax.fori_loop(
                        0, 16, jacc,
                        tuple(jnp.zeros((16,), f32) for _ in range(16)))
                    row = jnp.full((16,), rowl[b], i32)
                    for k in range(16):
                        plsc.addupdate_scatter(
                            stg, [row, jnp.full((16,), 16 * k, i32) + il],
                            accs[k])
                return 0

            lax.fori_loop(0, (c1 - c0 + _KB - 1) // _KB, cbatch, 0)
            pltpu.async_copy(stg, out_s.at[pl.ds(16 * g, 16)], semF)

        par = g - (g // 2) * 2

        @pl.when(par == 0)
        def _():
            do(stg0, semF0)

        @pl.when(par == 1)
        def _():
            do(stg1, semF1)
        return 0

    lax.fori_loop(blk0, blk1, cblk, 0)

    def drain(g):
        par = g - (g // 2) * 2

        @pl.when((g >= blk0) & (par == 0))
        def _():
            pltpu.make_async_copy(stg0, out_s.at[pl.ds(16 * g, 16)],
                                  semF0).wait()

        @pl.when((g >= blk0) & (par == 1))
        def _():
            pltpu.make_async_copy(stg1, out_s.at[pl.ds(16 * g, 16)],
                                  semF1).wait()

    drain(blk1 - 1)
    drain(blk1 - 2)


def _edge_stage_sc(pre, src_ctx, dst_ctx, ft):
    (src_pad, ns_pad, cns_pad, cst, cnd, cln, blk, tcs, teb) = pre
    i32 = jnp.int32
    f32 = jnp.float32
    mesh = plsc.VectorSubcoreMesh(core_axis_name="c", subcore_axis_name="s")
    cp = pltpu.CompilerParams()
    if "needs_layout_passes" in pltpu.CompilerParams.__dataclass_fields__:
        cp = dataclasses.replace(cp, needs_layout_passes=False)
    k = pl.kernel(
        _edge_sc_body,
        out_type=jax.ShapeDtypeStruct((N, D), f32),
        mesh=mesh,
        compiler_params=cp,
        scratch_types=[
            pltpu.VMEM((_NSCAP,), i32),
            pltpu.VMEM((_NSCAP,), i32),
            pltpu.VMEM((_SRCSL,), i32),
            pltpu.VMEM((_CTSL,), i32),
            pltpu.VMEM((_CTSL,), i32),
            pltpu.VMEM((_CTSL,), i32),
            pltpu.VMEM((_ECAP,), f32),
            pltpu.VMEM((_KB, 16, D), f32),
            pltpu.VMEM((_KB, D), f32),
            pltpu.VMEM((_KB, 16), i32),
            pltpu.VMEM((16, D), f32),
            pltpu.VMEM((16, D), f32),
            pltpu.VMEM((_NW + 8,), i32),
            pltpu.VMEM((_NW + 8,), i32),
            pltpu.VMEM((_NW + 8,), i32),
            pltpu.SemaphoreType.DMA,
            pltpu.SemaphoreType.DMA,
            pltpu.SemaphoreType.DMA,
        ],
    )
    return k(src_ctx, dst_ctx, ft, src_pad, ns_pad, cns_pad, cst, cnd, cln,
             blk, tcs, teb)


def kernel(situation_x, cmd_h, cmd_out, edge_index, cmdLength,
           graph_membership, Wmap, W1, W2, W3, b3, W4, W5, W6, W7, W8, W9,
           W10, W11, W11b, W12, initMem):
    f32 = jnp.float32
    src = edge_index[0]
    dst = edge_index[1]
    onehot = (graph_membership[:, None] == jnp.arange(B)[None, :]).astype(f32)

    cmd_all = _compute_cmd_all(cmd_h, cmd_out, cmdLength, W1, W2, W3, b3)
    x_loc = _matmul(situation_x, Wmap)
    pre = _edge_preprocess(edge_index)

    x_ctx = jnp.broadcast_to(initMem, (N, D))
    s_prev = jnp.zeros((N, D), f32)
    w11_t = jnp.eye(D, dtype=f32)
    w11b_t = jnp.zeros((D, D), f32)
    for t in range(T):
        x_ctx, src_ctx, dst_ctx, ft = _dense_stage(
            x_loc, x_ctx, s_prev, onehot, cmd_all[t], w11_t, w11b_t,
            W4, W5, W6, W7, W8, W9, W10)
        s_prev = _edge_stage_sc(pre, src_ctx, dst_ctx, ft)
        w11_t, w11b_t = W11, W11b
    return _final_stage(x_loc, x_ctx, s_prev, W11, W11b, W12)
